# Initial kernel scaffold; baseline (speedup 1.0000x reference)
import jax
import jax.numpy as jnp
from jax.experimental import pallas as pl


def kernel(index, target, token_emb_table):
    raise NotImplementedError("write your pallas kernel here")


# speedup vs baseline: 1.4127x; 1.4127x over previous
import functools

import jax
import jax.numpy as jnp
from jax import lax
from jax.experimental import pallas as pl
from jax.experimental.pallas import tpu as pltpu
from jax.experimental.pallas import tpu_sc as plsc

VOCAB = 1000
N_TOK = 1024 * 50
NUM_CORES = 2
NUM_SUBCORES = 16
NW = NUM_CORES * NUM_SUBCORES
PER_W = N_TOK // NW
CHUNK = 64
NCHUNK = PER_W // CHUNK

_mesh = plsc.VectorSubcoreMesh(
    core_axis_name="c", subcore_axis_name="s",
    num_cores=NUM_CORES, num_subcores=NUM_SUBCORES)


def _lse_body(tab_ref, out_ref):
    x = tab_ref[...]
    m = jnp.max(x, axis=1)
    s = jnp.sum(jnp.exp(x - m[:, None]), axis=1)
    out_ref[...] = m + jnp.log(s)


_lse_call = pl.pallas_call(
    _lse_body,
    out_shape=jax.ShapeDtypeStruct((VOCAB,), jnp.float32),
)


@functools.partial(
    pl.kernel,
    out_type=(jax.ShapeDtypeStruct((N_TOK, VOCAB), jnp.float32),
              jax.ShapeDtypeStruct((NW, 16), jnp.float32)),
    mesh=_mesh,
    compiler_params=pltpu.CompilerParams(use_tc_tiling_on_sc=False,
                                         needs_layout_passes=False),
    scratch_types=[
        pltpu.VMEM((PER_W,), jnp.int32),
        pltpu.VMEM((PER_W,), jnp.int32),
        pltpu.VMEM((VOCAB,), jnp.float32),
        pltpu.VMEM((CHUNK, VOCAB), jnp.float32),
        pltpu.VMEM((16,), jnp.float32),
        pltpu.SemaphoreType.DMA,
    ],
)
def _sc_gather(table_hbm, idx_hbm, tgt_hbm, lse_hbm, out_hbm, part_hbm,
               idx_v, tgt_v, lse_v, rows_v, acc_v, sem):
    wid = lax.axis_index("s") * NUM_CORES + lax.axis_index("c")
    base = wid * PER_W
    pltpu.sync_copy(idx_hbm.at[pl.ds(base, PER_W)], idx_v)
    pltpu.sync_copy(tgt_hbm.at[pl.ds(base, PER_W)], tgt_v)
    pltpu.sync_copy(lse_hbm, lse_v)
    lane = lax.iota(jnp.int32, 16)

    def chunk_body(c, acc):
        off = c * CHUNK
        pltpu.async_copy(
            table_hbm.at[idx_v.at[pl.ds(off, CHUNK)]], rows_v, sem).wait()
        for j in range(CHUNK // 16):
            sl = pl.ds(off + j * 16, 16)
            ivals = idx_v[sl]
            tvals = tgt_v[sl]
            rvals = plsc.load_gather(rows_v, [lane + j * 16, tvals])
            lvals = plsc.load_gather(lse_v, [ivals])
            acc = acc + (lvals - rvals)
        pltpu.sync_copy(rows_v, out_hbm.at[pl.ds(base + off, CHUNK)])
        return acc

    acc = lax.fori_loop(0, NCHUNK, chunk_body,
                        jnp.zeros((16,), jnp.float32))
    acc_v[...] = acc
    pltpu.sync_copy(acc_v, part_hbm.at[wid])


def _loss_body(p_ref, out_ref):
    out_ref[...] = jnp.sum(p_ref[...], keepdims=True).reshape(1, 1) * (1.0 / N_TOK)


_loss_call = pl.pallas_call(
    _loss_body,
    out_shape=jax.ShapeDtypeStruct((1, 1), jnp.float32),
)


def kernel(index, target, token_emb_table):
    b, t = index.shape
    idx = index.reshape(-1).astype(jnp.int32)
    tgt = target.reshape(-1).astype(jnp.int32)
    lse = _lse_call(token_emb_table)
    logits_flat, partials = _sc_gather(token_emb_table, idx, tgt, lse)
    loss = _loss_call(partials)[0, 0]
    return logits_flat.reshape(b, t, VOCAB), loss


# speedup vs baseline: 1.4369x; 1.0172x over previous
import functools

import jax
import jax.numpy as jnp
from jax import lax
from jax.experimental import pallas as pl
from jax.experimental.pallas import tpu as pltpu
from jax.experimental.pallas import tpu_sc as plsc

VOCAB = 1000
N_TOK = 1024 * 50
NUM_CORES = 2
NUM_SUBCORES = 16
NW = NUM_CORES * NUM_SUBCORES
PER_W = N_TOK // NW
CHUNK = 32
NCHUNK = PER_W // CHUNK
NBUF = 3

_mesh = plsc.VectorSubcoreMesh(
    core_axis_name="c", subcore_axis_name="s",
    num_cores=NUM_CORES, num_subcores=NUM_SUBCORES)


def _lse_body(tab_ref, out_ref):
    x = tab_ref[...]
    m = jnp.max(x, axis=1)
    s = jnp.sum(jnp.exp(x - m[:, None]), axis=1)
    out_ref[...] = m + jnp.log(s)


_lse_call = pl.pallas_call(
    _lse_body,
    out_shape=jax.ShapeDtypeStruct((VOCAB,), jnp.float32),
)


@functools.partial(
    pl.kernel,
    out_type=(jax.ShapeDtypeStruct((N_TOK, VOCAB), jnp.float32),
              jax.ShapeDtypeStruct((NW, 16), jnp.float32)),
    mesh=_mesh,
    compiler_params=pltpu.CompilerParams(use_tc_tiling_on_sc=False,
                                         needs_layout_passes=False),
    scratch_types=[
        pltpu.VMEM((PER_W,), jnp.int32),
        pltpu.VMEM((PER_W,), jnp.int32),
        pltpu.VMEM((VOCAB,), jnp.float32),
        pltpu.VMEM((CHUNK, VOCAB), jnp.float32),
        pltpu.VMEM((CHUNK, VOCAB), jnp.float32),
        pltpu.VMEM((CHUNK, VOCAB), jnp.float32),
        pltpu.VMEM((16,), jnp.float32),
        pltpu.SemaphoreType.DMA,
        pltpu.SemaphoreType.DMA,
        pltpu.SemaphoreType.DMA,
        pltpu.SemaphoreType.DMA,
        pltpu.SemaphoreType.DMA,
        pltpu.SemaphoreType.DMA,
    ],
)
def _sc_gather(table_hbm, idx_hbm, tgt_hbm, lse_hbm, out_hbm, part_hbm,
               idx_v, tgt_v, lse_v, rows0, rows1, rows2, acc_v,
               gin0, gin1, gin2, gout0, gout1, gout2):
    bufs = (rows0, rows1, rows2)
    gins = (gin0, gin1, gin2)
    gouts = (gout0, gout1, gout2)
    wid = lax.axis_index("s") * NUM_CORES + lax.axis_index("c")
    base = wid * PER_W
    pltpu.sync_copy(idx_hbm.at[pl.ds(base, PER_W)], idx_v)
    pltpu.sync_copy(tgt_hbm.at[pl.ds(base, PER_W)], tgt_v)
    pltpu.sync_copy(lse_hbm, lse_v)
    lane = lax.iota(jnp.int32, 16)

    def g_desc(c, u):
        return pltpu.make_async_copy(
            table_hbm.at[idx_v.at[pl.ds(c * CHUNK, CHUNK)]], bufs[u], gins[u])

    def s_desc(c, u):
        return pltpu.make_async_copy(
            bufs[u], out_hbm.at[pl.ds(base + c * CHUNK, CHUNK)], gouts[u])

    def compute(c, u, acc):
        off = c * CHUNK
        for j in range(CHUNK // 16):
            sl = pl.ds(off + j * 16, 16)
            ivals = idx_v[sl]
            tvals = tgt_v[sl]
            rvals = plsc.load_gather(bufs[u], [lane + j * 16, tvals])
            lvals = plsc.load_gather(lse_v, [ivals])
            acc = acc + (lvals - rvals)
        return acc

    def step(c, u, acc, wait_sc, issue_next):
        un = (u + 1) % NBUF
        if issue_next:
            if wait_sc:
                s_desc(c - 2, un).wait()
            g_desc(c + 1, un).start()
        g_desc(c, u).wait()
        s_desc(c, u).start()
        return compute(c, u, acc)

    acc = jnp.zeros((16,), jnp.float32)
    g_desc(0, 0).start()
    acc = step(0, 0, acc, wait_sc=False, issue_next=True)
    acc = step(1, 1, acc, wait_sc=False, issue_next=True)
    acc = step(2, 2, acc, wait_sc=True, issue_next=True)

    def ring_body(p, acc):
        c0 = 3 * p
        for u in range(NBUF):
            acc = step(c0 + u, u, acc, wait_sc=True, issue_next=True)
        return acc

    acc = lax.fori_loop(1, (NCHUNK - 2) // 3, ring_body, acc)

    acc = step(NCHUNK - 2, 0, acc, wait_sc=True, issue_next=True)
    acc = step(NCHUNK - 1, 1, acc, wait_sc=False, issue_next=False)
    s_desc(NCHUNK - 3, 2).wait()
    s_desc(NCHUNK - 2, 0).wait()
    s_desc(NCHUNK - 1, 1).wait()
    acc_v[...] = acc
    pltpu.sync_copy(acc_v, part_hbm.at[wid])


def _loss_body(p_ref, out_ref):
    out_ref[...] = jnp.sum(p_ref[...], keepdims=True).reshape(1, 1) * (1.0 / N_TOK)


_loss_call = pl.pallas_call(
    _loss_body,
    out_shape=jax.ShapeDtypeStruct((1, 1), jnp.float32),
)


def kernel(index, target, token_emb_table):
    b, t = index.shape
    idx = index.reshape(-1).astype(jnp.int32)
    tgt = target.reshape(-1).astype(jnp.int32)
    lse = _lse_call(token_emb_table)
    logits_flat, partials = _sc_gather(token_emb_table, idx, tgt, lse)
    loss = _loss_call(partials)[0, 0]
    return logits_flat.reshape(b, t, VOCAB), loss


# speedup vs baseline: 1.4415x; 1.0032x over previous
import functools

import jax
import jax.numpy as jnp
from jax import lax
from jax.experimental import pallas as pl
from jax.experimental.pallas import tpu as pltpu
from jax.experimental.pallas import tpu_sc as plsc

VOCAB = 1000
NBATCH = 1024
T = 50
N_TOK = NBATCH * T
NUM_CORES = 2
NUM_SUBCORES = 16
NW = NUM_CORES * NUM_SUBCORES
B_PER_W = NBATCH // NW
PER_W = B_PER_W * T
CHUNK = T
NCHUNK = B_PER_W
NBUF = 2
TPAD = 64
PER_W_PAD = B_PER_W * TPAD

_mesh = plsc.VectorSubcoreMesh(
    core_axis_name="c", subcore_axis_name="s",
    num_cores=NUM_CORES, num_subcores=NUM_SUBCORES)


def _lse_body(tab_ref, out_ref):
    x = tab_ref[...]
    m = jnp.max(x, axis=1)
    s = jnp.sum(jnp.exp(x - m[:, None]), axis=1)
    out_ref[...] = m + jnp.log(s)


_lse_call = pl.pallas_call(
    _lse_body,
    out_shape=jax.ShapeDtypeStruct((VOCAB,), jnp.float32),
)


@functools.partial(
    pl.kernel,
    out_type=(jax.ShapeDtypeStruct((NBATCH, T, VOCAB), jnp.float32),
              jax.ShapeDtypeStruct((NW, 16), jnp.float32)),
    mesh=_mesh,
    compiler_params=pltpu.CompilerParams(use_tc_tiling_on_sc=False,
                                         needs_layout_passes=False),
    scratch_types=[
        pltpu.VMEM((PER_W_PAD,), jnp.int32),
        pltpu.VMEM((PER_W_PAD,), jnp.int32),
        pltpu.VMEM((VOCAB,), jnp.float32),
        pltpu.VMEM((CHUNK, VOCAB), jnp.float32),
        pltpu.VMEM((CHUNK, VOCAB), jnp.float32),
        pltpu.VMEM((16,), jnp.float32),
        pltpu.SemaphoreType.DMA,
        pltpu.SemaphoreType.DMA,
        pltpu.SemaphoreType.DMA,
        pltpu.SemaphoreType.DMA,
    ],
)
def _sc_gather(table_hbm, idx_hbm, tgt_hbm, lse_hbm, out_hbm, part_hbm,
               idx_v, tgt_v, lse_v, rows0, rows1, acc_v,
               gin0, gin1, gout0, gout1):
    bufs = (rows0, rows1)
    gins = (gin0, gin1)
    gouts = (gout0, gout1)
    wid = lax.axis_index("s") * NUM_CORES + lax.axis_index("c")
    base = wid * PER_W_PAD
    batch0 = wid * B_PER_W
    pltpu.sync_copy(idx_hbm.at[pl.ds(base, PER_W_PAD)], idx_v)
    pltpu.sync_copy(tgt_hbm.at[pl.ds(base, PER_W_PAD)], tgt_v)
    pltpu.sync_copy(lse_hbm, lse_v)
    lane = lax.iota(jnp.int32, 16)
    tailmask = (lane < T - 48).astype(jnp.float32)
    tailrows = jnp.minimum(lane + 48, T - 1)

    def g_desc(c, u):
        return pltpu.make_async_copy(
            table_hbm.at[idx_v.at[pl.ds(c * TPAD, CHUNK)]], bufs[u], gins[u])

    def s_desc(c, u):
        return pltpu.make_async_copy(
            bufs[u], out_hbm.at[batch0 + c], gouts[u])

    def compute(c, u, acc):
        off = c * TPAD
        for start, rbase, msk in ((0, lane, None), (16, lane + 16, None),
                                  (32, lane + 32, None),
                                  (48, tailrows, tailmask)):
            sl = pl.ds(off + start, 16)
            ivals = idx_v[sl]
            tvals = tgt_v[sl]
            rvals = plsc.load_gather(bufs[u], [rbase, tvals])
            lvals = plsc.load_gather(lse_v, [ivals])
            d = lvals - rvals
            acc = acc + (d * msk if msk is not None else d)
        return acc

    def step(c, u, acc, wait_sc, issue_next):
        un = (u + 1) % NBUF
        if issue_next:
            if wait_sc:
                s_desc(c - 1, un).wait()
            g_desc(c + 1, un).start()
        g_desc(c, u).wait()
        s_desc(c, u).start()
        return compute(c, u, acc)

    acc = jnp.zeros((16,), jnp.float32)
    g_desc(0, 0).start()
    acc = step(0, 0, acc, wait_sc=False, issue_next=True)
    acc = step(1, 1, acc, wait_sc=True, issue_next=True)

    def ring_body(p, acc):
        c0 = 2 * p
        for u in range(NBUF):
            acc = step(c0 + u, u, acc, wait_sc=True, issue_next=True)
        return acc

    acc = lax.fori_loop(1, NCHUNK // 2 - 1, ring_body, acc)

    acc = step(NCHUNK - 2, 0, acc, wait_sc=True, issue_next=True)
    acc = step(NCHUNK - 1, 1, acc, wait_sc=False, issue_next=False)
    s_desc(NCHUNK - 2, 0).wait()
    s_desc(NCHUNK - 1, 1).wait()
    acc_v[...] = acc
    pltpu.sync_copy(acc_v, part_hbm.at[wid])


def _loss_body(p_ref, out_ref):
    out_ref[...] = jnp.sum(p_ref[...], keepdims=True).reshape(1, 1) * (1.0 / N_TOK)


_loss_call = pl.pallas_call(
    _loss_body,
    out_shape=jax.ShapeDtypeStruct((1, 1), jnp.float32),
)


def kernel(index, target, token_emb_table):
    pad = ((0, 0), (0, TPAD - T))
    idx = jnp.pad(index.astype(jnp.int32), pad).reshape(-1)
    tgt = jnp.pad(target.astype(jnp.int32), pad).reshape(-1)
    lse = _lse_call(token_emb_table)
    logits, partials = _sc_gather(token_emb_table, idx, tgt, lse)
    loss = _loss_call(partials)[0, 0]
    return logits, loss
